# Initial kernel scaffold; baseline (speedup 1.0000x reference)
import jax
import jax.numpy as jnp
from jax.experimental import pallas as pl


def kernel(x, pos, node_attr, edge_index, edge_attr, batch, W_q, W_si, W1_k, W2_k, W1_v, W2_v, W_dot):
    raise NotImplementedError("write your pallas kernel here")


# speedup vs baseline: 3.4362x; 3.4362x over previous
import functools
import numpy as np
import jax
import jax.numpy as jnp
from jax import lax
from jax.experimental import pallas as pl
from jax.experimental.pallas import tpu as pltpu
from jax.experimental.pallas import tpu_sc as plsc

_N = 10000
_E = 160000
_C = 32
_A = 8
_O = 32
_Q = 16
_K = 16
_NB = 8
_EA = 16
_MAX_R = 6.0
_SILU_NORM = 1.6768
_EMB_C = 1.14136 * float(np.exp(2.0))
_SQRT_NB = float(np.sqrt(_NB))
_INV_S24 = 1.0 / float(np.sqrt(_NB + _EA))
_INV_S128 = 1.0 / float(np.sqrt(128.0))
_INV_S32 = 1.0 / float(np.sqrt(_C))
_QD_SCALE = 1.0 / (_C * float(np.sqrt(_Q * _K)))
_SI_SCALE = 1.0 / float(np.sqrt(_C * _A))

_BN = 1000
_BE = 1000

_NW = 32
_EPW = _E // _NW
_GCH = 1000
_NCH = _EPW // _GCH
_NPT = _N // 16


def _sus(x):
    safe = jnp.where(x > 0.0, x, 1.0)
    return jnp.where(x > 0.0, jnp.exp(-1.0 / safe), 0.0)


def _silu(x):
    return x / (1.0 + jnp.exp(-x))


def _node_prep_body(x_ref, na_ref, wq_ref, wdot_ref, wsi_ref, qd_ref, si_ref):
    x = x_ref[...]
    na = na_ref[...]
    wqd = jnp.dot(wq_ref[...], wdot_ref[...], preferred_element_type=jnp.float32)
    qd_ref[...] = jnp.dot(x, wqd, preferred_element_type=jnp.float32) * _QD_SCALE
    xa = jnp.concatenate([x * na[:, a:a + 1] for a in range(_A)], axis=1)
    si_ref[...] = jnp.dot(xa, wsi_ref[...], preferred_element_type=jnp.float32) * _SI_SCALE


def _node_prep(x, node_attr, W_q, W_dot, wsi2):
    grid = (_N // _BN,)
    return pl.pallas_call(
        _node_prep_body,
        grid=grid,
        in_specs=[
            pl.BlockSpec((_BN, _C), lambda i: (i, 0)),
            pl.BlockSpec((_BN, _A), lambda i: (i, 0)),
            pl.BlockSpec((_C, _Q), lambda i: (0, 0)),
            pl.BlockSpec((_Q, _K), lambda i: (0, 0)),
            pl.BlockSpec((_A * _C, _O), lambda i: (0, 0)),
        ],
        out_specs=[
            pl.BlockSpec((_BN, _K), lambda i: (i, 0)),
            pl.BlockSpec((_BN, _O), lambda i: (i, 0)),
        ],
        out_shape=[
            jax.ShapeDtypeStruct((_N, _K), jnp.float32),
            jax.ShapeDtypeStruct((_N, _O), jnp.float32),
        ],
    )(x, node_attr, W_q, W_dot, wsi2)


@functools.cache
def _build_sc_gather():
  mesh = plsc.VectorSubcoreMesh(core_axis_name="c", subcore_axis_name="s")

  @functools.partial(
    pl.kernel,
    mesh=mesh,
    out_type=(
        jax.ShapeDtypeStruct((_E, _C), jnp.float32),
        jax.ShapeDtypeStruct((_E, _K), jnp.float32),
        jax.ShapeDtypeStruct((_E,), jnp.float32),
    ),
    scratch_types=[
        pltpu.VMEM((_EPW + 16,), jnp.int32),
        pltpu.VMEM((_EPW + 16,), jnp.int32),
        pltpu.VMEM((_GCH, _C), jnp.float32),
        pltpu.VMEM((_GCH, _K), jnp.float32),
        pltpu.VMEM((_EPW + 16,), jnp.float32),
        pltpu.VMEM((_N,), jnp.float32),
        pltpu.VMEM((_N,), jnp.float32),
        pltpu.VMEM((_N,), jnp.float32),
        pltpu.SemaphoreType.DMA,
        pltpu.SemaphoreType.DMA,
    ],
    compiler_params=pltpu.CompilerParams(needs_layout_passes=False, use_tc_tiling_on_sc=False),
  )
  def _sc_gather(src_h, dst_h, xt, qdt, px_h, py_h, pz_h, xs_out, qd_out,
                 l2_out, src_v, dst_v, xbuf, qbuf, l2v, px, py, pz, sem1,
                 sem2):
    wid = lax.axis_index("s") * 2 + lax.axis_index("c")
    base = wid * _EPW
    pltpu.sync_copy(src_h.at[pl.ds(base, _EPW)], src_v.at[pl.ds(0, _EPW)])
    pltpu.sync_copy(dst_h.at[pl.ds(base, _EPW)], dst_v.at[pl.ds(0, _EPW)])
    pltpu.sync_copy(px_h, px)
    pltpu.sync_copy(py_h, py)
    pltpu.sync_copy(pz_h, pz)

    def chunk_body(ci, carry):
        off = ci * _GCH
        pltpu.async_copy(xt.at[src_v.at[pl.ds(off, _GCH)]], xbuf, sem1).wait()
        pltpu.sync_copy(xbuf, xs_out.at[pl.ds(base + off, _GCH)])
        pltpu.async_copy(qdt.at[dst_v.at[pl.ds(off, _GCH)]], qbuf, sem2).wait()
        pltpu.sync_copy(qbuf, qd_out.at[pl.ds(base + off, _GCH)])
        return carry

    lax.fori_loop(0, _NCH, chunk_body, 0)

    lane = lax.iota(jnp.int32, 16)

    def l2_body(j, carry):
        o = j * 16
        ok = (o + lane) < _EPW
        s16 = jnp.where(ok, src_v[pl.ds(o, 16)], 0)
        d16 = jnp.where(ok, dst_v[pl.ds(o, 16)], 0)
        dx = plsc.load_gather(px, [s16]) - plsc.load_gather(px, [d16])
        dy = plsc.load_gather(py, [s16]) - plsc.load_gather(py, [d16])
        dz = plsc.load_gather(pz, [s16]) - plsc.load_gather(pz, [d16])
        l2v[pl.ds(o, 16)] = dx * dx + dy * dy + dz * dz + 1e-24
        return carry

    lax.fori_loop(0, (_EPW + 15) // 16, l2_body, 0)
    pltpu.sync_copy(l2v.at[pl.ds(0, _EPW)], l2_out.at[pl.ds(base, _EPW)])

  return _sc_gather


def _edge_body(xs_ref, qd_ref, l2_ref, ea_ref, w1k_ref, w2k_ref, w1v_ref,
               w2v_ref, out_ref):
    f32 = jnp.float32
    xs = xs_ref[...]
    elen = jnp.sqrt(l2_ref[...])

    jcol = lax.broadcasted_iota(jnp.int32, (_BE, _NB), 1).astype(f32)
    vals = (jcol + 1.0) * (_MAX_R / (_NB + 1))
    diff = (elen - vals) * ((_NB + 1) / _MAX_R)
    emb = (_EMB_C * _SQRT_NB) * _sus(diff + 1.0) * _sus(1.0 - diff)
    ed = jnp.concatenate([emb, ea_ref[...]], axis=1)

    hk = _silu(jnp.dot(ed, w1k_ref[...], preferred_element_type=f32) * _INV_S24) * _SILU_NORM
    wk2 = jnp.dot(hk, w2k_ref[...], preferred_element_type=f32) * _INV_S128
    hv = _silu(jnp.dot(ed, w1v_ref[...], preferred_element_type=f32) * _INV_S24) * _SILU_NORM
    wv2 = jnp.dot(hv, w2v_ref[...], preferred_element_type=f32) * _INV_S128

    rk_r = lax.broadcasted_iota(jnp.int32, (_C, _C * _K), 0)
    rk_c = lax.broadcasted_iota(jnp.int32, (_C, _C * _K), 1)
    Rk = (rk_c // _K == rk_r).astype(f32)
    sk_r = lax.broadcasted_iota(jnp.int32, (_C * _K, _K), 0)
    sk_c = lax.broadcasted_iota(jnp.int32, (_C * _K, _K), 1)
    Sk = (sk_r % _K == sk_c).astype(f32)
    xs_k = jnp.dot(xs, Rk, preferred_element_type=f32)
    kraw = jnp.dot(wk2 * xs_k, Sk, preferred_element_type=f32)

    rv_r = lax.broadcasted_iota(jnp.int32, (_C, _C * _O), 0)
    rv_c = lax.broadcasted_iota(jnp.int32, (_C, _C * _O), 1)
    Rv = (rv_c // _O == rv_r).astype(f32)
    sv_r = lax.broadcasted_iota(jnp.int32, (_C * _O, _O), 0)
    sv_c = lax.broadcasted_iota(jnp.int32, (_C * _O, _O), 1)
    Sv = (sv_r % _O == sv_c).astype(f32)
    xs_v = jnp.dot(xs, Rv, preferred_element_type=f32)
    vraw = jnp.dot(wv2 * xs_v, Sv, preferred_element_type=f32)

    temp = jnp.sum(qd_ref[...] * kraw, axis=1, keepdims=True)
    ewc = _sus(10.0 * (1.0 - elen * (1.0 / _MAX_R)))
    t2 = ewc * temp
    expv = jnp.exp(t2)
    sexp = jnp.exp(0.5 * t2)
    num = sexp * vraw * _INV_S32
    out_ref[...] = jnp.concatenate(
        [num, expv, jnp.zeros((_BE, 15), f32)], axis=1)


def _edge_pass(xs_g, qd_g, l2, edge_attr, W1_k, W2_k, W1_v, W2_v):
    grid = (_E // _BE,)
    return pl.pallas_call(
        _edge_body,
        grid=grid,
        in_specs=[
            pl.BlockSpec((_BE, _C), lambda i: (i, 0)),
            pl.BlockSpec((_BE, _K), lambda i: (i, 0)),
            pl.BlockSpec((_BE, 1), lambda i: (i, 0)),
            pl.BlockSpec((_BE, _EA), lambda i: (i, 0)),
            pl.BlockSpec((_NB + _EA, 128), lambda i: (0, 0)),
            pl.BlockSpec((128, _C * _K), lambda i: (0, 0)),
            pl.BlockSpec((_NB + _EA, 128), lambda i: (0, 0)),
            pl.BlockSpec((128, _C * _O), lambda i: (0, 0)),
        ],
        out_specs=pl.BlockSpec((_BE, 48), lambda i: (i, 0)),
        out_shape=jax.ShapeDtypeStruct((_E, 48), jnp.float32),
    )(xs_g, qd_g, l2, edge_attr, W1_k, W2_k, W1_v, W2_v)


@functools.cache
def _build_sc_scatter():
  mesh = plsc.VectorSubcoreMesh(core_axis_name="c", subcore_axis_name="s")

  @functools.partial(
    pl.kernel,
    mesh=mesh,
    out_type=jax.ShapeDtypeStruct((2, _N, 48), jnp.float32),
    scratch_types=[
        pltpu.VMEM((_GCH,), jnp.int32),
        pltpu.VMEM((_GCH, 48), jnp.float32),
        pltpu.VMEM((_NPT, 48), jnp.float32),
        pltpu.VMEM_SHARED((_N, 48), jnp.float32),
        pltpu.SemaphoreType.DMA,
    ],
    compiler_params=pltpu.CompilerParams(needs_layout_passes=False, use_tc_tiling_on_sc=False),
  )
  def _sc_scatter(dst_h, rows_h, out_h, dstc, rowsv, zb, table, sem):
    cid = lax.axis_index("c")
    sid = lax.axis_index("s")
    wid = sid * 2 + cid
    base = wid * _EPW

    zero16 = jnp.zeros((16,), jnp.float32)

    def zb_body(i, carry):
        r = i // 3
        c = (i % 3) * 16
        zb[r, pl.ds(c, 16)] = zero16
        return carry

    lax.fori_loop(0, _NPT * 3, zb_body, 0)
    pltpu.sync_copy(zb, table.at[pl.ds(sid * _NPT, _NPT)])
    plsc.subcore_barrier()

    def chunk_body(ci, carry):
        off = base + ci * _GCH
        pltpu.sync_copy(dst_h.at[pl.ds(off, _GCH)], dstc)
        pltpu.sync_copy(rows_h.at[pl.ds(off, _GCH)], rowsv)
        pltpu.sync_copy(rowsv, table.at[dstc], add=True)
        return carry

    lax.fori_loop(0, _NCH, chunk_body, 0)
    plsc.subcore_barrier()
    pltpu.sync_copy(table.at[pl.ds(sid * _NPT, _NPT)],
                    out_h.at[cid, pl.ds(sid * _NPT, _NPT)])

  return _sc_scatter


def _combine_body(s0_ref, s1_ref, si_ref, out_ref):
    s = s0_ref[0] + s1_ref[0]
    z = s[:, 32:33]
    zz = jnp.where(z == 0.0, 1.0, z)
    out_ref[...] = si_ref[...] + s[:, 0:_O] * lax.rsqrt(zz)


def _combine(s48, si):
    grid = (_N // _BN,)
    return pl.pallas_call(
        _combine_body,
        grid=grid,
        in_specs=[
            pl.BlockSpec((1, _BN, 48), lambda i: (0, i, 0)),
            pl.BlockSpec((1, _BN, 48), lambda i: (1, i, 0)),
            pl.BlockSpec((_BN, _O), lambda i: (i, 0)),
        ],
        out_specs=pl.BlockSpec((_BN, _O), lambda i: (i, 0)),
        out_shape=jax.ShapeDtypeStruct((_N, _O), jnp.float32),
    )(s48, s48, si)


def kernel(x, pos, node_attr, edge_index, edge_attr, batch, W_q, W_si,
           W1_k, W2_k, W1_v, W2_v, W_dot):
    px = jnp.asarray(pos[:, 0], jnp.float32)
    py = jnp.asarray(pos[:, 1], jnp.float32)
    pz = jnp.asarray(pos[:, 2], jnp.float32)
    wsi2 = jnp.transpose(W_si, (1, 0, 2)).reshape(_A * _C, _O)
    src = edge_index[0]
    dst = edge_index[1]
    qd, si = _node_prep(x, node_attr, W_q, W_dot, wsi2)
    xs_g, qd_g, l2 = _build_sc_gather()(src, dst, x, qd, px, py, pz)
    out48 = _edge_pass(xs_g, qd_g, l2.reshape(_E, 1), edge_attr,
                       W1_k, W2_k, W1_v, W2_v)
    s48 = _build_sc_scatter()(dst, out48)
    return _combine(s48, si)
